# baseline (speedup 1.0000x reference)
import functools

import numpy as np

import jax
import jax.numpy as jnp
from jax import lax
from jax.experimental import pallas as pl
from jax.experimental.pallas import tpu as pltpu
from jax.experimental.pallas import tpu_sc as plsc

N_NODES = 10000
N_EDGES = 320000
D = 128

NC = 2
NS = 16
NW = NC * NS
CH = 32
EPW = 10240
NCHUNK = EPW // CH
IBLK = 32
NBLK = NCHUNK // IBLK
NPAD = 10240
TRASH = N_NODES + 8
SLICE = NPAD // NS


def _tables_body(feat_ref, w_ref, b_ref, a_ref, bt_ref):
    w1 = w_ref[:, :D]
    w2 = w_ref[:, D:]
    f = feat_ref[...]
    dn = (((1,), (1,)), ((), ()))
    a_ref[...] = lax.dot_general(f, w1 - w2, dn,
                                 preferred_element_type=jnp.float32) + b_ref[...]
    bt_ref[...] = lax.dot_general(f, w2, dn,
                                  preferred_element_type=jnp.float32)


def _make_tables(feature, W, b):
    return pl.pallas_call(
        _tables_body,
        out_shape=(
            jax.ShapeDtypeStruct((N_NODES, D), jnp.float32),
            jax.ShapeDtypeStruct((N_NODES, D), jnp.float32),
        ),
    )(feature, W, b.reshape(1, D))


def _edge_body(a_hbm, b_hbm, src_hbm, dst_hbm, psum_hbm, pcnt_hbm,
               idx_src, idx_dst, buf0a, buf0b, buf1a, buf1b, hist,
               acc, sem0a, sem0b, sem1a, sem1b):
    buf0 = (buf0a, buf0b)
    buf1 = (buf1a, buf1b)
    sems = ((sem0a, sem0b), (sem1a, sem1b))
    c = lax.axis_index("c")
    s = lax.axis_index("s")
    w = c * NS + s

    zeros16 = jnp.zeros((16,), jnp.float32)
    e0 = jnp.where(lax.iota(jnp.int32, 16) == 0, 1.0, 0.0)

    def _fill_buf(i, _):
        for j in range(D // 16):
            buf0a[i, pl.ds(j * 16, 16)] = zeros16
        return 0
    lax.fori_loop(0, CH, _fill_buf, 0)

    def _fill_hist(i, _):
        hist[pl.ds(i * 16, 16)] = zeros16
        return 0
    lax.fori_loop(0, NPAD // 16, _fill_hist, 0)

    base = s * SLICE
    for k in range(SLICE // CH):
        pltpu.sync_copy(buf0a, acc.at[pl.ds(base + k * CH, CH)])
    plsc.subcore_barrier()

    bufs = (buf0, buf1)

    def _gather(ci, k):
        buf_a, buf_b = bufs[k]
        pltpu.async_copy(a_hbm.at[idx_dst.at[ci]], buf_a, sems[k][0])
        pltpu.async_copy(b_hbm.at[idx_src.at[ci]], buf_b, sems[k][1])

    def _consume(ci, k):
        buf_a, buf_b = bufs[k]
        pltpu.make_async_copy(a_hbm.at[pl.ds(0, CH)], buf_a, sems[k][0]).wait()
        pltpu.make_async_copy(b_hbm.at[pl.ds(0, CH)], buf_b, sems[k][1]).wait()

        def _row(i, _):
            for j in range(D // 16):
                sl = pl.ds(j * 16, 16)
                buf_a[i, sl] = jnp.maximum(buf_a[i, sl] + buf_b[i, sl], 0.0)
            return 0
        lax.fori_loop(0, CH, _row, 0)

        def _cnt(k2, _):
            idxv = idx_dst[ci, pl.ds(k2 * 16, 16)]
            for l in range(16):
                hsl = pl.ds(idxv[l], 16)
                hist[hsl] = hist[hsl] + e0
            return 0
        lax.fori_loop(0, CH // 16, _cnt, 0)

        pltpu.sync_copy(buf_a, acc.at[idx_dst.at[ci]], add=True)

    for bi in range(NBLK):
        pltpu.sync_copy(src_hbm.at[w, bi], idx_src)
        pltpu.sync_copy(dst_hbm.at[w, bi], idx_dst)
        _gather(0, 0)

        def _pair(k2, _):
            ci = k2 * 2
            _gather(ci + 1, 1)
            _consume(ci, 0)
            _gather(ci + 2, 0)
            _consume(ci + 1, 1)
            return 0
        lax.fori_loop(0, IBLK // 2 - 1, _pair, 0)
        _gather(IBLK - 1, 1)
        _consume(IBLK - 2, 0)
        _consume(IBLK - 1, 1)

    plsc.subcore_barrier()

    for k in range(SLICE // CH):
        off = base + k * CH
        pltpu.sync_copy(acc.at[pl.ds(off, CH)], buf0a)
        pltpu.sync_copy(buf0a, psum_hbm.at[c, pl.ds(off, CH)])
    pltpu.sync_copy(hist, pcnt_hbm.at[w])


@functools.partial(
    pl.kernel,
    out_type=(
        jax.ShapeDtypeStruct((NC, NPAD, D), jnp.float32),
        jax.ShapeDtypeStruct((NW, NPAD), jnp.float32),
    ),
    mesh=plsc.VectorSubcoreMesh(core_axis_name="c", subcore_axis_name="s"),
    scratch_types=[
        pltpu.VMEM((IBLK, CH), jnp.int32),
        pltpu.VMEM((IBLK, CH), jnp.int32),
        pltpu.VMEM((CH, D), jnp.float32),
        pltpu.VMEM((CH, D), jnp.float32),
        pltpu.VMEM((CH, D), jnp.float32),
        pltpu.VMEM((CH, D), jnp.float32),
        pltpu.VMEM((NPAD,), jnp.float32),
        pltpu.VMEM_SHARED((NPAD, D), jnp.float32),
        pltpu.SemaphoreType.DMA,
        pltpu.SemaphoreType.DMA,
        pltpu.SemaphoreType.DMA,
        pltpu.SemaphoreType.DMA,
    ],
)
def _edge_kernel(a_hbm, b_hbm, src_hbm, dst_hbm, psum_hbm, pcnt_hbm,
                 idx_src, idx_dst, buf0a, buf0b, buf1a, buf1b, hist,
                 acc, sem0a, sem0b, sem1a, sem1b):
    _edge_body(a_hbm, b_hbm, src_hbm, dst_hbm, psum_hbm, pcnt_hbm,
               idx_src, idx_dst, buf0a, buf0b, buf1a, buf1b, hist,
               acc, sem0a, sem0b, sem1a, sem1b)


def _final_body(psum_ref, pcnt_ref, out_ref):
    tot = psum_ref[0, :N_NODES] + psum_ref[1, :N_NODES]
    cnt = jnp.sum(pcnt_ref[...], axis=0)
    cntcol = cnt[:N_NODES].reshape(N_NODES, 1)
    out_ref[...] = tot / jnp.maximum(cntcol, 1.0)


def _finalize(psum, pcnt):
    return pl.pallas_call(
        _final_body,
        out_shape=jax.ShapeDtypeStruct((N_NODES, D), jnp.float32),
    )(psum, pcnt)


def kernel(feature, edge_index, W, b):
    a_tab, b_tab = _make_tables(feature, W, b)
    npad_e = NW * EPW - N_EDGES
    src = jnp.concatenate(
        [edge_index[0], jnp.zeros((npad_e,), jnp.int32)])
    dst = jnp.concatenate(
        [edge_index[1], jnp.full((npad_e,), TRASH, jnp.int32)])
    src4 = src.reshape(NW, NBLK, IBLK, CH)
    dst4 = dst.reshape(NW, NBLK, IBLK, CH)
    psum, pcnt = _edge_kernel(a_tab, b_tab, src4, dst4)
    return _finalize(psum, pcnt)


# speedup vs baseline: 1.5029x; 1.5029x over previous
import functools

import numpy as np

import jax
import jax.numpy as jnp
from jax import lax
from jax.experimental import pallas as pl
from jax.experimental.pallas import tpu as pltpu
from jax.experimental.pallas import tpu_sc as plsc

N_NODES = 10000
N_EDGES = 320000
D = 128

NC = 2
NS = 16
NW = NC * NS
CH = 80
EPW = 10000
NCHUNK = EPW // CH
IBLK = 25
NBLK = NCHUNK // IBLK
NPAD = 10240
TRASH = N_NODES + 8
SLICE = NPAD // NS


def _tables_body(feat_ref, w_ref, b_ref, a_ref, bt_ref):
    w1 = w_ref[:, :D]
    w2 = w_ref[:, D:]
    f = feat_ref[...]
    dn = (((1,), (1,)), ((), ()))
    a_ref[...] = lax.dot_general(f, w1 - w2, dn,
                                 preferred_element_type=jnp.float32) + b_ref[...]
    bt_ref[...] = lax.dot_general(f, w2, dn,
                                  preferred_element_type=jnp.float32)


def _make_tables(feature, W, b):
    return pl.pallas_call(
        _tables_body,
        out_shape=(
            jax.ShapeDtypeStruct((N_NODES, D), jnp.float32),
            jax.ShapeDtypeStruct((N_NODES, D), jnp.float32),
        ),
    )(feature, W, b.reshape(1, D))


def _edge_body(a_hbm, b_hbm, src_hbm, dst_hbm, psum_hbm, pcnt_hbm,
               idx_src, idx_dst, buf0a, buf0b, hist,
               acc, sem0a, sem0b):
    buf0 = (buf0a, buf0b)
    bufs = (buf0,)
    sems = ((sem0a, sem0b),)
    c = lax.axis_index("c")
    s = lax.axis_index("s")
    w = c * NS + s

    zeros16 = jnp.zeros((16,), jnp.float32)
    e0 = jnp.where(lax.iota(jnp.int32, 16) == 0, 1.0, 0.0)

    def _fill_buf(i, _):
        for j in range(D // 16):
            buf0a[i, pl.ds(j * 16, 16)] = zeros16
        return 0
    lax.fori_loop(0, CH, _fill_buf, 0)

    def _fill_hist(i, _):
        hist[pl.ds(i * 16, 16)] = zeros16
        return 0
    lax.fori_loop(0, NPAD // 16, _fill_hist, 0)

    base = s * SLICE
    for k in range(SLICE // CH):
        pltpu.sync_copy(buf0a, acc.at[pl.ds(base + k * CH, CH)])
    plsc.subcore_barrier()


    def _gather(ci, k):
        buf_a, buf_b = bufs[k]
        pltpu.async_copy(a_hbm.at[idx_dst.at[ci]], buf_a, sems[k][0])
        pltpu.async_copy(b_hbm.at[idx_src.at[ci]], buf_b, sems[k][1])

    def _consume(ci, k):
        buf_a, buf_b = bufs[k]
        pltpu.make_async_copy(a_hbm.at[pl.ds(0, CH)], buf_a, sems[k][0]).wait()
        pltpu.make_async_copy(b_hbm.at[pl.ds(0, CH)], buf_b, sems[k][1]).wait()

        def _row(i, _):
            for j in range(D // 16):
                sl = pl.ds(j * 16, 16)
                buf_a[i, sl] = jnp.maximum(buf_a[i, sl] + buf_b[i, sl], 0.0)
            return 0
        lax.fori_loop(0, CH, _row, 0)

        def _cnt(k2, _):
            idxv = idx_dst[ci, pl.ds(k2 * 16, 16)]
            for l in range(16):
                hsl = pl.ds(idxv[l], 16)
                hist[hsl] = hist[hsl] + e0
            return 0
        lax.fori_loop(0, CH // 16, _cnt, 0)

        pltpu.sync_copy(buf_a, acc.at[idx_dst.at[ci]], add=True)

    for bi in range(NBLK):
        pltpu.sync_copy(src_hbm.at[w, bi], idx_src)
        pltpu.sync_copy(dst_hbm.at[w, bi], idx_dst)

        def _one(ci, _):
            _gather(ci, 0)
            _consume(ci, 0)
            return 0
        lax.fori_loop(0, IBLK, _one, 0)

    plsc.subcore_barrier()

    for k in range(SLICE // CH):
        off = base + k * CH
        pltpu.sync_copy(acc.at[pl.ds(off, CH)], buf0a)
        pltpu.sync_copy(buf0a, psum_hbm.at[c, pl.ds(off, CH)])
    pltpu.sync_copy(hist, pcnt_hbm.at[w])


@functools.partial(
    pl.kernel,
    out_type=(
        jax.ShapeDtypeStruct((NC, NPAD, D), jnp.float32),
        jax.ShapeDtypeStruct((NW, NPAD), jnp.float32),
    ),
    mesh=plsc.VectorSubcoreMesh(core_axis_name="c", subcore_axis_name="s"),
    scratch_types=[
        pltpu.VMEM((IBLK, CH), jnp.int32),
        pltpu.VMEM((IBLK, CH), jnp.int32),
        pltpu.VMEM((CH, D), jnp.float32),
        pltpu.VMEM((CH, D), jnp.float32),
        pltpu.VMEM((NPAD,), jnp.float32),
        pltpu.VMEM_SHARED((NPAD, D), jnp.float32),
        pltpu.SemaphoreType.DMA,
        pltpu.SemaphoreType.DMA,
    ],
)
def _edge_kernel(a_hbm, b_hbm, src_hbm, dst_hbm, psum_hbm, pcnt_hbm,
                 idx_src, idx_dst, buf0a, buf0b, hist,
                 acc, sem0a, sem0b):
    _edge_body(a_hbm, b_hbm, src_hbm, dst_hbm, psum_hbm, pcnt_hbm,
               idx_src, idx_dst, buf0a, buf0b, hist,
               acc, sem0a, sem0b)


def _final_body(psum_ref, pcnt_ref, out_ref):
    tot = psum_ref[0, :N_NODES] + psum_ref[1, :N_NODES]
    cnt = jnp.sum(pcnt_ref[...], axis=0)
    cntcol = cnt[:N_NODES].reshape(N_NODES, 1)
    out_ref[...] = tot / jnp.maximum(cntcol, 1.0)


def _finalize(psum, pcnt):
    return pl.pallas_call(
        _final_body,
        out_shape=jax.ShapeDtypeStruct((N_NODES, D), jnp.float32),
    )(psum, pcnt)


def kernel(feature, edge_index, W, b):
    a_tab, b_tab = _make_tables(feature, W, b)
    src4 = edge_index[0].reshape(NW, NBLK, IBLK, CH)
    dst4 = edge_index[1].reshape(NW, NBLK, IBLK, CH)
    psum, pcnt = _edge_kernel(a_tab, b_tab, src4, dst4)
    return _finalize(psum, pcnt)
